# Initial kernel scaffold; baseline (speedup 1.0000x reference)
import jax
import jax.numpy as jnp
from jax.experimental import pallas as pl


def kernel(pos, faces, face_features, W1, b1, W2, b2):
    raise NotImplementedError("write your pallas kernel here")


# speedup vs baseline: 6.8480x; 6.8480x over previous
import functools

import jax
import jax.numpy as jnp
from jax import lax
from jax.experimental import pallas as pl
from jax.experimental.pallas import tpu as pltpu
from jax.experimental.pallas import tpu_sc as plsc

NC = 2
NS = 16
NW = NC * NS
L = 16

_MESH = plsc.VectorSubcoreMesh(core_axis_name="c", subcore_axis_name="s",
                               num_cores=NC, num_subcores=NS)
_SC_PARAMS = pltpu.CompilerParams(needs_layout_passes=False)


def _gather_body(n_nodes, n_faces, fpt, ch, pos_hbm, fidx_hbm, out_hbm,
                 pos_v, idx_v, stg_v):
    wid = lax.axis_index("s") * NC + lax.axis_index("c")
    base_f = wid * fpt
    lane = lax.iota(jnp.int32, L)
    lane16 = lane * 16

    def zero(i, _):
        stg_v[pl.ds(i * L, L)] = jnp.zeros((L,), jnp.float32)
        return 0
    lax.fori_loop(0, (ch * 16) // L, zero, 0)

    pltpu.sync_copy(pos_hbm, pos_v)

    def chunk(k, _):
        cb = base_f + k * ch
        for r in range(3):
            pltpu.sync_copy(fidx_hbm.at[pl.ds(r * n_faces + cb, ch)],
                            idx_v.at[pl.ds(r * ch, ch)])

        def group(g, _):
            sbase = lane16 + g * (16 * L)
            for r in range(3):
                iv = idx_v[pl.ds(r * ch + g * L, L)]
                iv4 = iv * 4
                for c in range(3):
                    v = plsc.load_gather(pos_v, [iv4 + c])
                    plsc.store_scatter(stg_v, [sbase + (3 * r + c)], v)
            return 0
        lax.fori_loop(0, ch // L, group, 0)
        pltpu.sync_copy(stg_v, out_hbm.at[pl.ds(cb * 16, ch * 16)])
        return 0
    lax.fori_loop(0, fpt // ch, chunk, 0)


def _sc_gather(pos4_flat, fidx, n_nodes, n_faces):
    fpt = n_faces // NW
    ch = 2000
    assert fpt % ch == 0
    k = pl.kernel(
        functools.partial(_gather_body, n_nodes, n_faces, fpt, ch),
        out_type=jax.ShapeDtypeStruct((n_faces * 16,), jnp.float32),
        mesh=_MESH,
        scratch_types=[
            pltpu.VMEM((4 * n_nodes,), jnp.float32),
            pltpu.VMEM((3 * ch,), jnp.int32),
            pltpu.VMEM((ch * 16,), jnp.float32),
        ],
        compiler_params=_SC_PARAMS,
    )
    return k(pos4_flat, fidx)


def _mlp_body(p16_ref, ff_ref, a_ref, w1f_ref, b1_ref, w2_ref, b2_ref,
              nff_ref, cor_ref):
    ff = ff_ref[...]
    base = jnp.dot(ff, w1f_ref[...], preferred_element_type=jnp.float32)
    base = base + b1_ref[...]
    pterm = jnp.dot(p16_ref[...], a_ref[...],
                    preferred_element_type=jnp.float32)
    acc = None
    for r in range(3):
        h = jnp.maximum(base + pterm[:, r * 128:(r + 1) * 128], 0.0)
        g = jnp.dot(h, w2_ref[...], preferred_element_type=jnp.float32)
        g = g + b2_ref[...]
        cor_ref[r] = g[:, 0:4]
        acc = g[:, 3:] if acc is None else acc + g[:, 3:]
    nff_ref[...] = acc * (1.0 / 3.0)


def _tc_mlp(p16, ff, a16, w1f, b1, w2, b2, n_faces, out_ch):
    ft = 1280
    assert n_faces % ft == 0
    grid = n_faces // ft
    return pl.pallas_call(
        _mlp_body,
        grid=(grid,),
        in_specs=[
            pl.BlockSpec((ft, 16), lambda i: (i, 0)),
            pl.BlockSpec((ft, 128), lambda i: (i, 0)),
            pl.BlockSpec((16, 384), lambda i: (0, 0)),
            pl.BlockSpec((128, 128), lambda i: (0, 0)),
            pl.BlockSpec((1, 128), lambda i: (0, 0)),
            pl.BlockSpec((128, 32), lambda i: (0, 0)),
            pl.BlockSpec((1, 32), lambda i: (0, 0)),
        ],
        out_specs=[
            pl.BlockSpec((ft, out_ch), lambda i: (i, 0)),
            pl.BlockSpec((3, ft, 4), lambda i: (0, i, 0)),
        ],
        out_shape=[
            jax.ShapeDtypeStruct((n_faces, out_ch), jnp.float32),
            jax.ShapeDtypeStruct((3, n_faces, 4), jnp.float32),
        ],
    )(p16, ff, a16, w1f, b1, w2, b2)


def _scatter_body(npad, ppt, ch, idx_hbm, c_hbm, part_hbm,
                  acc_v, idx_v, cv_v, tmp_v, shared_v):
    cid = lax.axis_index("c")
    sid = lax.axis_index("s")
    wid = sid * NC + cid
    pbase0 = wid * ppt
    lane = lax.iota(jnp.int32, L)
    lane4 = lane * 4
    ones = jnp.ones((L,), jnp.float32)

    nwords = npad * 4

    def zero(i, _):
        acc_v[pl.ds(i * L, L)] = jnp.zeros((L,), jnp.float32)
        return 0
    lax.fori_loop(0, nwords // L, zero, 0)

    def chunk(k, _):
        pb = pbase0 + k * ch
        pltpu.sync_copy(idx_hbm.at[pl.ds(pb, ch)], idx_v)
        pltpu.sync_copy(c_hbm.at[pl.ds(pb * 4, ch * 4)], cv_v)

        def group(g, _):
            iv = idx_v[pl.ds(g * L, L)]
            iv4 = iv * 4
            gb = lane4 + g * (4 * L)
            for c in range(3):
                vals = plsc.load_gather(cv_v, [gb + c])
                plsc.addupdate_scatter(acc_v, [iv4 + c], vals)
            plsc.addupdate_scatter(acc_v, [iv4 + 3], ones)
            return 0
        lax.fori_loop(0, ch // L, group, 0)
        return 0
    lax.fori_loop(0, ppt // ch, chunk, 0)

    pltpu.sync_copy(acc_v, shared_v.at[pl.ds(sid * nwords, nwords)])
    plsc.subcore_barrier()

    wpt = nwords // NS
    off = sid * wpt
    red = acc_v
    pltpu.sync_copy(shared_v.at[pl.ds(off, wpt)], red.at[pl.ds(0, wpt)])
    for t in range(1, NS):
        pltpu.sync_copy(shared_v.at[pl.ds(t * nwords + off, wpt)], tmp_v)

        def add(i, _):
            red[pl.ds(i * L, L)] = red[pl.ds(i * L, L)] + tmp_v[pl.ds(i * L, L)]
            return 0
        lax.fori_loop(0, wpt // L, add, 0)
    pltpu.sync_copy(red.at[pl.ds(0, wpt)],
                    part_hbm.at[pl.ds(cid * nwords + off, wpt)])


def _sc_scatter(idx_flat, c_flat, npad, n_pairs):
    ppt = n_pairs // NW
    ch = 2000
    assert ppt % ch == 0
    nwords = npad * 4
    k = pl.kernel(
        functools.partial(_scatter_body, npad, ppt, ch),
        out_type=jax.ShapeDtypeStruct((NC * nwords,), jnp.float32),
        mesh=_MESH,
        scratch_types=[
            pltpu.VMEM((nwords,), jnp.float32),
            pltpu.VMEM((ch,), jnp.int32),
            pltpu.VMEM((ch * 4,), jnp.float32),
            pltpu.VMEM((nwords // NS,), jnp.float32),
            pltpu.VMEM_SHARED((NS * nwords,), jnp.float32),
        ],
        compiler_params=_SC_PARAMS,
    )
    return k(idx_flat, c_flat)


def _fin_body(n_nodes, part_ref, pos_ref, dpos_ref, npos_ref):
    s = part_ref[0] + part_ref[1]
    sums = s[:n_nodes, 0:3]
    cnt = s[:n_nodes, 3:4]
    delta = sums / jnp.maximum(cnt, 1.0)
    dpos_ref[...] = delta
    npos_ref[...] = pos_ref[...] + delta


def _tc_finalize(part, pos, n_nodes, npad):
    return pl.pallas_call(
        functools.partial(_fin_body, n_nodes),
        in_specs=[
            pl.BlockSpec((2, npad, 4), lambda: (0, 0, 0)),
            pl.BlockSpec((n_nodes, 3), lambda: (0, 0)),
        ],
        out_specs=[
            pl.BlockSpec((n_nodes, 3), lambda: (0, 0)),
            pl.BlockSpec((n_nodes, 3), lambda: (0, 0)),
        ],
        out_shape=[
            jax.ShapeDtypeStruct((n_nodes, 3), jnp.float32),
            jax.ShapeDtypeStruct((n_nodes, 3), jnp.float32),
        ],
    )(part, pos)


def kernel(pos, faces, face_features, W1, b1, W2, b2):
    n_nodes, _ = pos.shape
    n_faces = faces.shape[0]
    out_ch = W2.shape[1] - 3

    fidx = faces.astype(jnp.int32).T.reshape(-1)
    pos4 = jnp.pad(pos, ((0, 0), (0, 1))).reshape(-1)

    p16_flat = _sc_gather(pos4, fidx, n_nodes, n_faces)
    p16 = p16_flat.reshape(n_faces, 16)

    wa, wb = W1[0:3], W1[3:6]
    s = wa + wb
    a0 = jnp.concatenate([-s, wa, wb], axis=0)
    a1 = jnp.concatenate([wb, -s, wa], axis=0)
    a2 = jnp.concatenate([wa, wb, -s], axis=0)
    a16 = jnp.pad(jnp.concatenate([a0, a1, a2], axis=1), ((0, 7), (0, 0)))

    nff, corners = _tc_mlp(p16, face_features, a16, W1[6:], b1[None],
                           W2, b2[None], n_faces, out_ch)

    npad = NS * ((n_nodes + NS * 8 - 1) // (NS * 8)) * 8
    c_flat = corners.reshape(-1)
    part = _sc_scatter(fidx, c_flat, npad, 3 * n_faces)

    dpos, npos = _tc_finalize(part.reshape(NC, npad, 4), pos, n_nodes, npad)
    return (dpos, npos, nff)


# speedup vs baseline: 8.6703x; 1.2661x over previous
import functools

import jax
import jax.numpy as jnp
from jax import lax
from jax.experimental import pallas as pl
from jax.experimental.pallas import tpu as pltpu
from jax.experimental.pallas import tpu_sc as plsc

NC = 2
NS = 16
NW = NC * NS
L = 16

_MESH = plsc.VectorSubcoreMesh(core_axis_name="c", subcore_axis_name="s",
                               num_cores=NC, num_subcores=NS)
_SC_PARAMS = pltpu.CompilerParams(needs_layout_passes=False)


def _gather_body(n_nodes, n_faces, fpt, ch, pos_hbm, fidx_hbm, out_hbm,
                 pos_v, idx_v, stg_v):
    wid = lax.axis_index("s") * NC + lax.axis_index("c")
    base_f = wid * fpt
    lane = lax.iota(jnp.int32, L)
    lane16 = lane * 16

    def zero(i, _):
        stg_v[pl.ds(i * L, L)] = jnp.zeros((L,), jnp.float32)
        return 0
    lax.fori_loop(0, (ch * 16) // L, zero, 0)

    pltpu.sync_copy(pos_hbm, pos_v)

    def chunk(k, _):
        cb = base_f + k * ch
        for r in range(3):
            pltpu.sync_copy(fidx_hbm.at[pl.ds(r * n_faces + cb, ch)],
                            idx_v.at[pl.ds(r * ch, ch)])

        def group(g, _):
            sbase = lane16 + g * (16 * L)
            for r in range(3):
                iv = idx_v[pl.ds(r * ch + g * L, L)]
                iv4 = iv * 4
                for c in range(3):
                    v = plsc.load_gather(pos_v, [iv4 + c])
                    plsc.store_scatter(stg_v, [sbase + (3 * r + c)], v)
            return 0
        lax.fori_loop(0, ch // L, group, 0)
        pltpu.sync_copy(stg_v, out_hbm.at[pl.ds(cb * 16, ch * 16)])
        return 0
    lax.fori_loop(0, fpt // ch, chunk, 0)


def _sc_gather(pos4_flat, fidx, n_nodes, n_faces):
    fpt = n_faces // NW
    ch = 2000
    assert fpt % ch == 0
    k = pl.kernel(
        functools.partial(_gather_body, n_nodes, n_faces, fpt, ch),
        out_type=jax.ShapeDtypeStruct((n_faces * 16,), jnp.float32),
        mesh=_MESH,
        scratch_types=[
            pltpu.VMEM((4 * n_nodes,), jnp.float32),
            pltpu.VMEM((3 * ch,), jnp.int32),
            pltpu.VMEM((ch * 16,), jnp.float32),
        ],
        compiler_params=_SC_PARAMS,
    )
    return k(pos4_flat, fidx)


def _mlp_body(ft, pp_ref, ff_ref, a_ref, w1f_ref, b1_ref, w2_ref, b2_ref,
              w2t_ref, b2t_ref, nff_ref, *cor_refs):
    ff = ff_ref[...]
    base = jnp.dot(ff, w1f_ref[...], preferred_element_type=jnp.float32)
    base = base + b1_ref[...]
    pt = jnp.dot(pp_ref[...], a_ref[...], preferred_element_type=jnp.float32)
    pterm = pt.reshape(ft, 384)
    acc = None
    for r in range(3):
        h = jnp.maximum(base + pterm[:, r * 128:(r + 1) * 128], 0.0)
        g = jnp.dot(h, w2_ref[...], preferred_element_type=jnp.float32)
        g = g + b2_ref[...]
        cpt = lax.dot_general(w2t_ref[...], h, (((1,), (1,)), ((), ())),
                              preferred_element_type=jnp.float32)
        cpt = cpt + b2t_ref[...]
        for c in range(3):
            cor_refs[3 * r + c][...] = cpt[c]
        acc = g[:, 3:] if acc is None else acc + g[:, 3:]
    nff_ref[...] = acc * (1.0 / 3.0)


def _tc_mlp(pp, ff, a_stack, w1f, b1, w2, b2, w2t, b2t, n_faces, out_ch):
    ft = 512
    assert n_faces % ft == 0
    grid = n_faces // ft
    return pl.pallas_call(
        functools.partial(_mlp_body, ft),
        grid=(grid,),
        in_specs=[
            pl.BlockSpec((ft * 16 // 128, 128), lambda i: (i, 0)),
            pl.BlockSpec((ft, 128), lambda i: (i, 0)),
            pl.BlockSpec((128, 3072), lambda i: (0, 0)),
            pl.BlockSpec((128, 128), lambda i: (0, 0)),
            pl.BlockSpec((1, 128), lambda i: (0, 0)),
            pl.BlockSpec((128, 32), lambda i: (0, 0)),
            pl.BlockSpec((1, 32), lambda i: (0, 0)),
            pl.BlockSpec((3, 128), lambda i: (0, 0)),
            pl.BlockSpec((3, 1), lambda i: (0, 0)),
        ],
        out_specs=[pl.BlockSpec((ft, out_ch), lambda i: (i, 0))] +
                  [pl.BlockSpec((ft,), lambda i: (i,)) for _ in range(9)],
        out_shape=[jax.ShapeDtypeStruct((n_faces, out_ch), jnp.float32)] +
                  [jax.ShapeDtypeStruct((n_faces,), jnp.float32)
                   for _ in range(9)],
    )(pp, ff, a_stack, w1f, b1, w2, b2, w2t, b2t)


def _scatter_body(npad, n_faces, fpt, ch, idx_hbm, c00, c01, c02, c10, c11,
                  c12, c20, c21, c22, part_hbm,
                  acc_v, idx_v, cx_v, cy_v, cz_v, tmp_v, shared_v):
    cid = lax.axis_index("c")
    sid = lax.axis_index("s")
    wid = sid * NC + cid
    fbase0 = wid * fpt
    ones = jnp.ones((L,), jnp.float32)

    nwords = npad * 4

    def zero(i, _):
        acc_v[pl.ds(i * L, L)] = jnp.zeros((L,), jnp.float32)
        return 0
    lax.fori_loop(0, nwords // L, zero, 0)

    carr = ((c00, c01, c02), (c10, c11, c12), (c20, c21, c22))
    for r in range(3):
        def chunk(k, _):
            fb = fbase0 + k * ch
            pltpu.sync_copy(idx_hbm.at[pl.ds(r * n_faces + fb, ch)], idx_v)
            pltpu.sync_copy(carr[r][0].at[pl.ds(fb, ch)], cx_v)
            pltpu.sync_copy(carr[r][1].at[pl.ds(fb, ch)], cy_v)
            pltpu.sync_copy(carr[r][2].at[pl.ds(fb, ch)], cz_v)

            def group(g, _):
                iv = idx_v[pl.ds(g * L, L)]
                iv4 = iv * 4
                plsc.addupdate_scatter(acc_v, [iv4], cx_v[pl.ds(g * L, L)])
                plsc.addupdate_scatter(acc_v, [iv4 + 1], cy_v[pl.ds(g * L, L)])
                plsc.addupdate_scatter(acc_v, [iv4 + 2], cz_v[pl.ds(g * L, L)])
                plsc.addupdate_scatter(acc_v, [iv4 + 3], ones)
                return 0
            lax.fori_loop(0, ch // L, group, 0)
            return 0
        lax.fori_loop(0, fpt // ch, chunk, 0)

    pltpu.sync_copy(acc_v, shared_v.at[pl.ds(sid * nwords, nwords)])
    plsc.subcore_barrier()

    wpt = nwords // NS
    off = sid * wpt
    red = acc_v
    pltpu.sync_copy(shared_v.at[pl.ds(off, wpt)], red.at[pl.ds(0, wpt)])
    for t in range(1, NS):
        pltpu.sync_copy(shared_v.at[pl.ds(t * nwords + off, wpt)], tmp_v)

        def add(i, _):
            red[pl.ds(i * L, L)] = red[pl.ds(i * L, L)] + tmp_v[pl.ds(i * L, L)]
            return 0
        lax.fori_loop(0, wpt // L, add, 0)
    pltpu.sync_copy(red.at[pl.ds(0, wpt)],
                    part_hbm.at[pl.ds(cid * nwords + off, wpt)])


def _sc_scatter(idx_flat, comps, npad, n_faces):
    fpt = n_faces // NW
    ch = 2000
    assert fpt % ch == 0
    nwords = npad * 4
    k = pl.kernel(
        functools.partial(_scatter_body, npad, n_faces, fpt, ch),
        out_type=jax.ShapeDtypeStruct((NC * nwords,), jnp.float32),
        mesh=_MESH,
        scratch_types=[
            pltpu.VMEM((nwords,), jnp.float32),
            pltpu.VMEM((ch,), jnp.int32),
            pltpu.VMEM((ch,), jnp.float32),
            pltpu.VMEM((ch,), jnp.float32),
            pltpu.VMEM((ch,), jnp.float32),
            pltpu.VMEM((nwords // NS,), jnp.float32),
            pltpu.VMEM_SHARED((NS * nwords,), jnp.float32),
        ],
        compiler_params=_SC_PARAMS,
    )
    return k(idx_flat, *comps)


def _fin_body(n_nodes, part_ref, pos_ref, dpos_ref, npos_ref):
    s = part_ref[0] + part_ref[1]
    sums = s[:n_nodes, 0:3]
    cnt = s[:n_nodes, 3:4]
    delta = sums / jnp.maximum(cnt, 1.0)
    dpos_ref[...] = delta
    npos_ref[...] = pos_ref[...] + delta


def _tc_finalize(part, pos, n_nodes, npad):
    return pl.pallas_call(
        functools.partial(_fin_body, n_nodes),
        in_specs=[
            pl.BlockSpec((2, npad, 4), lambda: (0, 0, 0)),
            pl.BlockSpec((n_nodes, 3), lambda: (0, 0)),
        ],
        out_specs=[
            pl.BlockSpec((n_nodes, 3), lambda: (0, 0)),
            pl.BlockSpec((n_nodes, 3), lambda: (0, 0)),
        ],
        out_shape=[
            jax.ShapeDtypeStruct((n_nodes, 3), jnp.float32),
            jax.ShapeDtypeStruct((n_nodes, 3), jnp.float32),
        ],
    )(part, pos)


def kernel(pos, faces, face_features, W1, b1, W2, b2):
    n_nodes, _ = pos.shape
    n_faces = faces.shape[0]
    out_ch = W2.shape[1] - 3

    fidx = faces.astype(jnp.int32).T.reshape(-1)
    pos4 = jnp.pad(pos, ((0, 0), (0, 1))).reshape(-1)

    p16_flat = _sc_gather(pos4, fidx, n_nodes, n_faces)
    pp = p16_flat.reshape(n_faces * 16 // 128, 128)

    wa, wb = W1[0:3], W1[3:6]
    s = wa + wb
    a0 = jnp.concatenate([-s, wa, wb], axis=0)
    a1 = jnp.concatenate([wb, -s, wa], axis=0)
    a2 = jnp.concatenate([wa, wb, -s], axis=0)
    a16 = jnp.pad(jnp.concatenate([a0, a1, a2], axis=1), ((0, 7), (0, 0)))
    a_stack = jnp.kron(jnp.eye(8, dtype=jnp.float32), a16)

    outs = _tc_mlp(pp, face_features, a_stack, W1[6:], b1[None],
                   W2, b2[None], W2[:, 0:3].T, b2[0:3].reshape(3, 1),
                   n_faces, out_ch)
    nff, comps = outs[0], outs[1:]

    npad = NS * ((n_nodes + NS * 8 - 1) // (NS * 8)) * 8
    part = _sc_scatter(fidx, comps, npad, n_faces)

    dpos, npos = _tc_finalize(part.reshape(NC, npad, 4), pos, n_nodes, npad)
    return (dpos, npos, nff)


# speedup vs baseline: 8.6880x; 1.0020x over previous
import functools

import jax
import jax.numpy as jnp
from jax import lax
from jax.experimental import pallas as pl
from jax.experimental.pallas import tpu as pltpu
from jax.experimental.pallas import tpu_sc as plsc

NC = 2
NS = 16
NW = NC * NS
L = 16

_MESH = plsc.VectorSubcoreMesh(core_axis_name="c", subcore_axis_name="s",
                               num_cores=NC, num_subcores=NS)
_SC_PARAMS = pltpu.CompilerParams(needs_layout_passes=False)


def _gather_body(n_nodes, n_faces, fpt, ch, pos_hbm, fidx_hbm, out_hbm,
                 pos_v, idx_v, stg_v):
    wid = lax.axis_index("s") * NC + lax.axis_index("c")
    base_f = wid * fpt
    lane = lax.iota(jnp.int32, L)
    lane16 = lane * 16

    def zero(i, _):
        stg_v[pl.ds(i * L, L)] = jnp.zeros((L,), jnp.float32)
        return 0
    lax.fori_loop(0, (ch * 16) // L, zero, 0)

    pltpu.sync_copy(pos_hbm, pos_v)

    def chunk(k, _):
        cb = base_f + k * ch
        for r in range(3):
            pltpu.sync_copy(fidx_hbm.at[pl.ds(r * n_faces + cb, ch)],
                            idx_v.at[pl.ds(r * ch, ch)])

        def group(g, _):
            sbase = lane16 + g * (16 * L)
            for r in range(3):
                iv = idx_v[pl.ds(r * ch + g * L, L)]
                iv4 = iv * 4
                for c in range(3):
                    v = plsc.load_gather(pos_v, [iv4 + c])
                    plsc.store_scatter(stg_v, [sbase + (3 * r + c)], v)
            return 0
        lax.fori_loop(0, ch // L, group, 0)
        pltpu.sync_copy(stg_v, out_hbm.at[pl.ds(cb * 16, ch * 16)])
        return 0
    lax.fori_loop(0, fpt // ch, chunk, 0)


def _sc_gather(pos4_flat, fidx, n_nodes, n_faces):
    fpt = n_faces // NW
    ch = 2000
    assert fpt % ch == 0
    k = pl.kernel(
        functools.partial(_gather_body, n_nodes, n_faces, fpt, ch),
        out_type=jax.ShapeDtypeStruct((n_faces * 16,), jnp.float32),
        mesh=_MESH,
        scratch_types=[
            pltpu.VMEM((4 * n_nodes,), jnp.float32),
            pltpu.VMEM((3 * ch,), jnp.int32),
            pltpu.VMEM((ch * 16,), jnp.float32),
        ],
        compiler_params=_SC_PARAMS,
    )
    return k(pos4_flat, fidx)


def _mlp_body(ft, pp_ref, ff_ref, a_ref, w1f_ref, b1_ref, w2_ref, b2_ref,
              w2t_ref, b2t_ref, nff_ref, *cor_refs):
    ff = ff_ref[...].astype(jnp.bfloat16)
    base = jnp.dot(ff, w1f_ref[...], preferred_element_type=jnp.float32)
    base = base + b1_ref[...]
    pt = jnp.dot(pp_ref[...].astype(jnp.bfloat16), a_ref[...],
                 preferred_element_type=jnp.float32)
    pterm = pt.reshape(ft, 384)
    acc = None
    for r in range(3):
        h = jnp.maximum(base + pterm[:, r * 128:(r + 1) * 128], 0.0)
        hb = h.astype(jnp.bfloat16)
        g = jnp.dot(hb, w2_ref[...], preferred_element_type=jnp.float32)
        g = g + b2_ref[...]
        cpt = lax.dot_general(w2t_ref[...], hb, (((1,), (1,)), ((), ())),
                              preferred_element_type=jnp.float32)
        cpt = cpt + b2t_ref[...]
        for c in range(3):
            cor_refs[3 * r + c][...] = cpt[c]
        acc = g[:, 3:] if acc is None else acc + g[:, 3:]
    nff_ref[...] = acc * (1.0 / 3.0)


def _tc_mlp(pp, ff, a_stack, w1f, b1, w2, b2, w2t, b2t, n_faces, out_ch):
    ft = 512
    assert n_faces % ft == 0
    grid = n_faces // ft
    return pl.pallas_call(
        functools.partial(_mlp_body, ft),
        grid=(grid,),
        in_specs=[
            pl.BlockSpec((ft * 16 // 128, 128), lambda i: (i, 0)),
            pl.BlockSpec((ft, 128), lambda i: (i, 0)),
            pl.BlockSpec((128, 3072), lambda i: (0, 0)),
            pl.BlockSpec((128, 128), lambda i: (0, 0)),
            pl.BlockSpec((1, 128), lambda i: (0, 0)),
            pl.BlockSpec((128, 32), lambda i: (0, 0)),
            pl.BlockSpec((1, 32), lambda i: (0, 0)),
            pl.BlockSpec((3, 128), lambda i: (0, 0)),
            pl.BlockSpec((3, 1), lambda i: (0, 0)),
        ],
        out_specs=[pl.BlockSpec((ft, out_ch), lambda i: (i, 0))] +
                  [pl.BlockSpec((ft,), lambda i: (i,)) for _ in range(9)],
        out_shape=[jax.ShapeDtypeStruct((n_faces, out_ch), jnp.float32)] +
                  [jax.ShapeDtypeStruct((n_faces,), jnp.float32)
                   for _ in range(9)],
    )(pp, ff, a_stack, w1f, b1, w2, b2, w2t, b2t)


def _scatter_body(npad, n_faces, fpt, ch, idx_hbm, c00, c01, c02, c10, c11,
                  c12, c20, c21, c22, part_hbm,
                  acc_v, idx_v, cx_v, cy_v, cz_v, tmp_v, shared_v):
    cid = lax.axis_index("c")
    sid = lax.axis_index("s")
    wid = sid * NC + cid
    fbase0 = wid * fpt
    ones = jnp.ones((L,), jnp.float32)

    nwords = npad * 4

    def zero(i, _):
        acc_v[pl.ds(i * L, L)] = jnp.zeros((L,), jnp.float32)
        return 0
    lax.fori_loop(0, nwords // L, zero, 0)

    carr = ((c00, c01, c02), (c10, c11, c12), (c20, c21, c22))
    for r in range(3):
        def chunk(k, _):
            fb = fbase0 + k * ch
            pltpu.sync_copy(idx_hbm.at[pl.ds(r * n_faces + fb, ch)], idx_v)
            pltpu.sync_copy(carr[r][0].at[pl.ds(fb, ch)], cx_v)
            pltpu.sync_copy(carr[r][1].at[pl.ds(fb, ch)], cy_v)
            pltpu.sync_copy(carr[r][2].at[pl.ds(fb, ch)], cz_v)

            def group(g, _):
                iv = idx_v[pl.ds(g * L, L)]
                iv4 = iv * 4
                plsc.addupdate_scatter(acc_v, [iv4], cx_v[pl.ds(g * L, L)])
                plsc.addupdate_scatter(acc_v, [iv4 + 1], cy_v[pl.ds(g * L, L)])
                plsc.addupdate_scatter(acc_v, [iv4 + 2], cz_v[pl.ds(g * L, L)])
                plsc.addupdate_scatter(acc_v, [iv4 + 3], ones)
                return 0
            lax.fori_loop(0, ch // L, group, 0)
            return 0
        lax.fori_loop(0, fpt // ch, chunk, 0)

    pltpu.sync_copy(acc_v, shared_v.at[pl.ds(sid * nwords, nwords)])
    plsc.subcore_barrier()

    wpt = nwords // NS
    off = sid * wpt
    red = acc_v
    pltpu.sync_copy(shared_v.at[pl.ds(off, wpt)], red.at[pl.ds(0, wpt)])
    for t in range(1, NS):
        pltpu.sync_copy(shared_v.at[pl.ds(t * nwords + off, wpt)], tmp_v)

        def add(i, _):
            red[pl.ds(i * L, L)] = red[pl.ds(i * L, L)] + tmp_v[pl.ds(i * L, L)]
            return 0
        lax.fori_loop(0, wpt // L, add, 0)
    pltpu.sync_copy(red.at[pl.ds(0, wpt)],
                    part_hbm.at[pl.ds(cid * nwords + off, wpt)])


def _sc_scatter(idx_flat, comps, npad, n_faces):
    fpt = n_faces // NW
    ch = 2000
    assert fpt % ch == 0
    nwords = npad * 4
    k = pl.kernel(
        functools.partial(_scatter_body, npad, n_faces, fpt, ch),
        out_type=jax.ShapeDtypeStruct((NC * nwords,), jnp.float32),
        mesh=_MESH,
        scratch_types=[
            pltpu.VMEM((nwords,), jnp.float32),
            pltpu.VMEM((ch,), jnp.int32),
            pltpu.VMEM((ch,), jnp.float32),
            pltpu.VMEM((ch,), jnp.float32),
            pltpu.VMEM((ch,), jnp.float32),
            pltpu.VMEM((nwords // NS,), jnp.float32),
            pltpu.VMEM_SHARED((NS * nwords,), jnp.float32),
        ],
        compiler_params=_SC_PARAMS,
    )
    return k(idx_flat, *comps)


def _fin_body(n_nodes, part_ref, pos_ref, dpos_ref, npos_ref):
    s = part_ref[0] + part_ref[1]
    sums = s[:n_nodes, 0:3]
    cnt = s[:n_nodes, 3:4]
    delta = sums / jnp.maximum(cnt, 1.0)
    dpos_ref[...] = delta
    npos_ref[...] = pos_ref[...] + delta


def _tc_finalize(part, pos, n_nodes, npad):
    return pl.pallas_call(
        functools.partial(_fin_body, n_nodes),
        in_specs=[
            pl.BlockSpec((2, npad, 4), lambda: (0, 0, 0)),
            pl.BlockSpec((n_nodes, 3), lambda: (0, 0)),
        ],
        out_specs=[
            pl.BlockSpec((n_nodes, 3), lambda: (0, 0)),
            pl.BlockSpec((n_nodes, 3), lambda: (0, 0)),
        ],
        out_shape=[
            jax.ShapeDtypeStruct((n_nodes, 3), jnp.float32),
            jax.ShapeDtypeStruct((n_nodes, 3), jnp.float32),
        ],
    )(part, pos)


def kernel(pos, faces, face_features, W1, b1, W2, b2):
    n_nodes, _ = pos.shape
    n_faces = faces.shape[0]
    out_ch = W2.shape[1] - 3

    fidx = faces.astype(jnp.int32).T.reshape(-1)
    pos4 = jnp.pad(pos, ((0, 0), (0, 1))).reshape(-1)

    p16_flat = _sc_gather(pos4, fidx, n_nodes, n_faces)
    pp = p16_flat.reshape(n_faces * 16 // 128, 128)

    wa, wb = W1[0:3], W1[3:6]
    s = wa + wb
    a0 = jnp.concatenate([-s, wa, wb], axis=0)
    a1 = jnp.concatenate([wb, -s, wa], axis=0)
    a2 = jnp.concatenate([wa, wb, -s], axis=0)
    a16 = jnp.pad(jnp.concatenate([a0, a1, a2], axis=1), ((0, 7), (0, 0)))
    a_stack = jnp.kron(jnp.eye(8, dtype=jnp.float32), a16)

    bf = jnp.bfloat16
    outs = _tc_mlp(pp, face_features, a_stack.astype(bf), W1[6:].astype(bf),
                   b1[None], W2.astype(bf), b2[None],
                   W2[:, 0:3].T.astype(bf), b2[0:3].reshape(3, 1),
                   n_faces, out_ch)
    nff, comps = outs[0], outs[1:]

    npad = NS * ((n_nodes + NS * 8 - 1) // (NS * 8)) * 8
    part = _sc_scatter(fidx, comps, npad, n_faces)

    dpos, npos = _tc_finalize(part.reshape(NC, npad, 4), pos, n_nodes, npad)
    return (dpos, npos, nff)


# speedup vs baseline: 9.6870x; 1.1150x over previous
import functools

import jax
import jax.numpy as jnp
from jax import lax
from jax.experimental import pallas as pl
from jax.experimental.pallas import tpu as pltpu
from jax.experimental.pallas import tpu_sc as plsc

NC = 2
NS = 16
NW = NC * NS
L = 16

_MESH = plsc.VectorSubcoreMesh(core_axis_name="c", subcore_axis_name="s",
                               num_cores=NC, num_subcores=NS)
_SC_PARAMS = pltpu.CompilerParams(needs_layout_passes=False)


_BS = 512


def _gather_body(n_nodes, n_faces, nblk, bpt, pos_hbm, fidx_hbm, out_hbm,
                 pos_v, idx0, idx1, stg0, stg1, sem_i0, sem_i1, sem_o0,
                 sem_o1):
    wid = lax.axis_index("s") * NC + lax.axis_index("c")
    blk0 = wid * bpt
    lane = lax.iota(jnp.int32, L)

    idxb = (idx0, idx1)
    stgb = (stg0, stg1)
    semi = (sem_i0, sem_i1)
    semo = (sem_o0, sem_o1)

    def zero(i, _):
        stg0[pl.ds(i * L, L)] = jnp.zeros((L,), jnp.float32)
        stg1[pl.ds(i * L, L)] = jnp.zeros((L,), jnp.float32)
        return 0
    lax.fori_loop(0, (_BS * 16) // L, zero, 0)

    pltpu.sync_copy(pos_hbm, pos_v)

    def fire_idx(ci):
        blk = jnp.minimum(blk0 + ci, nblk - 1)
        fb = blk * _BS
        b = idxb[ci % 2]
        for r in range(3):
            pltpu.async_copy(fidx_hbm.at[pl.ds(r * n_faces + fb, _BS)],
                             b.at[pl.ds(r * _BS, _BS)], semi[ci % 2])

    def wait_idx(ci):
        b = idxb[ci % 2]
        for r in range(3):
            pltpu.make_async_copy(fidx_hbm.at[pl.ds(0, _BS)],
                                  b.at[pl.ds(r * _BS, _BS)],
                                  semi[ci % 2]).wait()

    fire_idx(0)
    for ci in range(bpt):
        blk = blk0 + ci
        valid = blk < nblk
        if ci + 1 < bpt:
            fire_idx(ci + 1)
        if ci - 2 >= 0:
            @pl.when((blk0 + ci - 2) < nblk)
            def _():
                pltpu.make_async_copy(
                    out_hbm.at[pl.ds(0, _BS * 16)], stgb[ci % 2],
                    semo[ci % 2]).wait()
        wait_idx(ci)
        ib = idxb[ci % 2]
        stg = stgb[ci % 2]

        def group(g, _):
            daddr = ((g % 4) * 16 + lane) * 128 + 16 * (g // 4)
            for r in range(3):
                iv = ib[pl.ds(r * _BS + g * L, L)]
                iv4 = iv * 4
                for c in range(3):
                    v = plsc.load_gather(pos_v, [iv4 + c])
                    plsc.store_scatter(stg, [daddr + (3 * r + c)], v)
            return 0
        lax.fori_loop(0, _BS // L, group, 0)

        @pl.when(valid)
        def _():
            pltpu.async_copy(stg, out_hbm.at[pl.ds(blk * (_BS * 16),
                                                   _BS * 16)], semo[ci % 2])
    for ci in (bpt - 2, bpt - 1):
        @pl.when((blk0 + ci) < nblk)
        def _():
            pltpu.make_async_copy(out_hbm.at[pl.ds(0, _BS * 16)],
                                  stgb[ci % 2], semo[ci % 2]).wait()


def _sc_gather(pos4_flat, fidx, n_nodes, n_faces):
    nblk = n_faces // _BS
    bpt = (nblk + NW - 1) // NW
    k = pl.kernel(
        functools.partial(_gather_body, n_nodes, n_faces, nblk, bpt),
        out_type=jax.ShapeDtypeStruct((n_faces * 16,), jnp.float32),
        mesh=_MESH,
        scratch_types=[
            pltpu.VMEM((4 * n_nodes,), jnp.float32),
            pltpu.VMEM((3 * _BS,), jnp.int32),
            pltpu.VMEM((3 * _BS,), jnp.int32),
            pltpu.VMEM((_BS * 16,), jnp.float32),
            pltpu.VMEM((_BS * 16,), jnp.float32),
            pltpu.SemaphoreType.DMA,
            pltpu.SemaphoreType.DMA,
            pltpu.SemaphoreType.DMA,
            pltpu.SemaphoreType.DMA,
        ],
        compiler_params=_SC_PARAMS,
    )
    return k(pos4_flat, fidx)


def _mlp_body(ft, pp_ref, ff_ref, a_ref, w1f_ref, b1_ref, w2_ref, b2_ref,
              w2t_ref, b2t_ref, nff_ref, *cor_refs):
    ff = ff_ref[...].astype(jnp.bfloat16)
    base = jnp.dot(ff, w1f_ref[...], preferred_element_type=jnp.float32)
    base = base + b1_ref[...]
    pt = jnp.dot(pp_ref[...].astype(jnp.bfloat16), a_ref[...],
                 preferred_element_type=jnp.float32)
    acc = None
    for r in range(3):
        hs = [base[64 * k:64 * (k + 1)] +
              pt[:, 384 * k + 128 * r:384 * k + 128 * r + 128]
              for k in range(8)]
        h = jnp.maximum(jnp.concatenate(hs, axis=0), 0.0)
        hb = h.astype(jnp.bfloat16)
        g = jnp.dot(hb, w2_ref[...], preferred_element_type=jnp.float32)
        g = g + b2_ref[...]
        cpt = lax.dot_general(w2t_ref[...], hb, (((1,), (1,)), ((), ())),
                              preferred_element_type=jnp.float32)
        cpt = cpt + b2t_ref[...]
        for c in range(3):
            cor_refs[3 * r + c][...] = cpt[c]
        acc = g[:, 3:] if acc is None else acc + g[:, 3:]
    nff_ref[...] = acc * (1.0 / 3.0)


def _tc_mlp(pp, ff, a_stack, w1f, b1, w2, b2, w2t, b2t, n_faces, out_ch):
    ft = 512
    assert n_faces % ft == 0
    grid = n_faces // ft
    return pl.pallas_call(
        functools.partial(_mlp_body, ft),
        grid=(grid,),
        in_specs=[
            pl.BlockSpec((ft * 16 // 128, 128), lambda i: (i, 0)),
            pl.BlockSpec((ft, 128), lambda i: (i, 0)),
            pl.BlockSpec((128, 3072), lambda i: (0, 0)),
            pl.BlockSpec((128, 128), lambda i: (0, 0)),
            pl.BlockSpec((1, 128), lambda i: (0, 0)),
            pl.BlockSpec((128, 32), lambda i: (0, 0)),
            pl.BlockSpec((1, 32), lambda i: (0, 0)),
            pl.BlockSpec((3, 128), lambda i: (0, 0)),
            pl.BlockSpec((3, 1), lambda i: (0, 0)),
        ],
        out_specs=[pl.BlockSpec((ft, out_ch), lambda i: (i, 0))] +
                  [pl.BlockSpec((ft,), lambda i: (i,)) for _ in range(9)],
        out_shape=[jax.ShapeDtypeStruct((n_faces, out_ch), jnp.float32)] +
                  [jax.ShapeDtypeStruct((n_faces,), jnp.float32)
                   for _ in range(9)],
    )(pp, ff, a_stack, w1f, b1, w2, b2, w2t, b2t)


def _scatter_body(npad, n_faces, fpt, ch, idx_hbm, c00, c01, c02, c10, c11,
                  c12, c20, c21, c22, part_hbm,
                  acc_v, idxb0, idxb1, buf0, buf1, tmp0, tmp1, shared_v,
                  sem_b0, sem_b1, sem_r0, sem_r1):
    cid = lax.axis_index("c")
    sid = lax.axis_index("s")
    wid = sid * NC + cid
    fbase0 = wid * fpt
    ones = jnp.ones((L,), jnp.float32)

    nwords = npad * 4

    def zero(i, _):
        acc_v[pl.ds(i * L, L)] = jnp.zeros((L,), jnp.float32)
        return 0
    lax.fori_loop(0, nwords // L, zero, 0)

    carr = ((c00, c01, c02), (c10, c11, c12), (c20, c21, c22))
    chunks = [(r, k) for r in range(3) for k in range(fpt // ch)]
    idxbs = (idxb0, idxb1)
    bufs = (buf0, buf1)
    sems = (sem_b0, sem_b1)

    def fire(ci):
        r, k = chunks[ci]
        fb = fbase0 + k * ch
        ib, b, s = idxbs[ci % 2], bufs[ci % 2], sems[ci % 2]
        pltpu.async_copy(idx_hbm.at[pl.ds(r * n_faces + fb, ch)], ib, s)
        for c in range(3):
            pltpu.async_copy(carr[r][c].at[pl.ds(fb, ch)],
                             b.at[pl.ds(c * ch, ch)], s)

    def wait(ci):
        ib, b, s = idxbs[ci % 2], bufs[ci % 2], sems[ci % 2]
        pltpu.make_async_copy(idx_hbm.at[pl.ds(0, ch)], ib, s).wait()
        for c in range(3):
            pltpu.make_async_copy(c00.at[pl.ds(0, ch)],
                                  b.at[pl.ds(c * ch, ch)], s).wait()

    fire(0)
    for ci in range(len(chunks)):
        if ci + 1 < len(chunks):
            fire(ci + 1)
        wait(ci)
        ib, b = idxbs[ci % 2], bufs[ci % 2]

        def group(g, _):
            iv = ib[pl.ds(g * L, L)]
            iv4 = iv * 4
            plsc.addupdate_scatter(acc_v, [iv4], b[pl.ds(g * L, L)])
            plsc.addupdate_scatter(acc_v, [iv4 + 1],
                                   b[pl.ds(ch + g * L, L)])
            plsc.addupdate_scatter(acc_v, [iv4 + 2],
                                   b[pl.ds(2 * ch + g * L, L)])
            plsc.addupdate_scatter(acc_v, [iv4 + 3], ones)
            return 0
        lax.fori_loop(0, ch // L, group, 0)

    pltpu.sync_copy(acc_v, shared_v.at[pl.ds(sid * nwords, nwords)])
    plsc.subcore_barrier()

    wpt = nwords // NS
    off = sid * wpt
    red = acc_v
    tb = (tmp0, tmp1)
    ts = (sem_r0, sem_r1)
    pltpu.sync_copy(shared_v.at[pl.ds(off, wpt)], red.at[pl.ds(0, wpt)])
    pltpu.async_copy(shared_v.at[pl.ds(nwords + off, wpt)], tmp1, sem_r1)
    for t in range(1, NS):
        if t + 1 < NS:
            pltpu.async_copy(shared_v.at[pl.ds((t + 1) * nwords + off, wpt)],
                             tb[(t + 1) % 2], ts[(t + 1) % 2])
        pltpu.make_async_copy(part_hbm.at[pl.ds(0, wpt)], tb[t % 2],
                              ts[t % 2]).wait()
        tv = tb[t % 2]

        def add(i, _):
            red[pl.ds(i * L, L)] = red[pl.ds(i * L, L)] + tv[pl.ds(i * L, L)]
            return 0
        lax.fori_loop(0, wpt // L, add, 0)
    pltpu.sync_copy(red.at[pl.ds(0, wpt)],
                    part_hbm.at[pl.ds(cid * nwords + off, wpt)])


def _sc_scatter(idx_flat, comps, npad, n_faces):
    fpt = n_faces // NW
    ch = 2000
    assert fpt % ch == 0
    nwords = npad * 4
    k = pl.kernel(
        functools.partial(_scatter_body, npad, n_faces, fpt, ch),
        out_type=jax.ShapeDtypeStruct((NC * nwords,), jnp.float32),
        mesh=_MESH,
        scratch_types=[
            pltpu.VMEM((nwords,), jnp.float32),
            pltpu.VMEM((ch,), jnp.int32),
            pltpu.VMEM((ch,), jnp.int32),
            pltpu.VMEM((3 * ch,), jnp.float32),
            pltpu.VMEM((3 * ch,), jnp.float32),
            pltpu.VMEM((nwords // NS,), jnp.float32),
            pltpu.VMEM((nwords // NS,), jnp.float32),
            pltpu.VMEM_SHARED((NS * nwords,), jnp.float32),
            pltpu.SemaphoreType.DMA,
            pltpu.SemaphoreType.DMA,
            pltpu.SemaphoreType.DMA,
            pltpu.SemaphoreType.DMA,
        ],
        compiler_params=_SC_PARAMS,
    )
    return k(idx_flat, *comps)


def _fin_body(n_nodes, part_ref, pos_ref, dpos_ref, npos_ref):
    s = part_ref[0] + part_ref[1]
    sums = s[:n_nodes, 0:3]
    cnt = s[:n_nodes, 3:4]
    delta = sums / jnp.maximum(cnt, 1.0)
    dpos_ref[...] = delta
    npos_ref[...] = pos_ref[...] + delta


def _tc_finalize(part, pos, n_nodes, npad):
    return pl.pallas_call(
        functools.partial(_fin_body, n_nodes),
        in_specs=[
            pl.BlockSpec((2, npad, 4), lambda: (0, 0, 0)),
            pl.BlockSpec((n_nodes, 3), lambda: (0, 0)),
        ],
        out_specs=[
            pl.BlockSpec((n_nodes, 3), lambda: (0, 0)),
            pl.BlockSpec((n_nodes, 3), lambda: (0, 0)),
        ],
        out_shape=[
            jax.ShapeDtypeStruct((n_nodes, 3), jnp.float32),
            jax.ShapeDtypeStruct((n_nodes, 3), jnp.float32),
        ],
    )(part, pos)


def kernel(pos, faces, face_features, W1, b1, W2, b2):
    n_nodes, _ = pos.shape
    n_faces = faces.shape[0]
    out_ch = W2.shape[1] - 3

    fidx = faces.astype(jnp.int32).T.reshape(-1)
    pos4 = jnp.pad(pos, ((0, 0), (0, 1))).reshape(-1)

    p16_flat = _sc_gather(pos4, fidx, n_nodes, n_faces)
    pp = p16_flat.reshape(n_faces * 16 // 128, 128)

    wa, wb = W1[0:3], W1[3:6]
    s = wa + wb
    a0 = jnp.concatenate([-s, wa, wb], axis=0)
    a1 = jnp.concatenate([wb, -s, wa], axis=0)
    a2 = jnp.concatenate([wa, wb, -s], axis=0)
    a16 = jnp.pad(jnp.concatenate([a0, a1, a2], axis=1), ((0, 7), (0, 0)))
    a_stack = jnp.kron(jnp.eye(8, dtype=jnp.float32), a16)

    bf = jnp.bfloat16
    outs = _tc_mlp(pp, face_features, a_stack.astype(bf), W1[6:].astype(bf),
                   b1[None], W2.astype(bf), b2[None],
                   W2[:, 0:3].T.astype(bf), b2[0:3].reshape(3, 1),
                   n_faces, out_ch)
    nff, comps = outs[0], outs[1:]

    npad = NS * ((n_nodes + NS * 8 - 1) // (NS * 8)) * 8
    part = _sc_scatter(fidx, comps, npad, n_faces)

    dpos, npos = _tc_finalize(part.reshape(NC, npad, 4), pos, n_nodes, npad)
    return (dpos, npos, nff)


# speedup vs baseline: 16.2682x; 1.6794x over previous
import functools

import jax
import jax.numpy as jnp
from jax import lax
from jax.experimental import pallas as pl
from jax.experimental.pallas import tpu as pltpu
from jax.experimental.pallas import tpu_sc as plsc

NC = 2
NS = 16
NW = NC * NS
L = 16

_MESH = plsc.VectorSubcoreMesh(core_axis_name="c", subcore_axis_name="s",
                               num_cores=NC, num_subcores=NS)
_SC_PARAMS = pltpu.CompilerParams(needs_layout_passes=False)


_BS = 512


def _gather_body(n_nodes, n_faces, nblk, bpt, pos_hbm, fidx_hbm, out_hbm,
                 pos_v, idx0, idx1, stg0, stg1, sem_i0, sem_i1, sem_o0,
                 sem_o1):
    wid = lax.axis_index("s") * NC + lax.axis_index("c")
    blk0 = wid * bpt
    lane = lax.iota(jnp.int32, L)

    idxb = (idx0, idx1)
    stgb = (stg0, stg1)
    semi = (sem_i0, sem_i1)
    semo = (sem_o0, sem_o1)

    @plsc.parallel_loop(0, (_BS * 16) // L, unroll=4)
    def zero(i):
        stg0[pl.ds(i * L, L)] = jnp.zeros((L,), jnp.float32)
        stg1[pl.ds(i * L, L)] = jnp.zeros((L,), jnp.float32)

    pltpu.sync_copy(pos_hbm, pos_v)

    def fire_idx(ci):
        blk = jnp.minimum(blk0 + ci, nblk - 1)
        fb = blk * _BS
        b = idxb[ci % 2]
        for r in range(3):
            pltpu.async_copy(fidx_hbm.at[pl.ds(r * n_faces + fb, _BS)],
                             b.at[pl.ds(r * _BS, _BS)], semi[ci % 2])

    def wait_idx(ci):
        b = idxb[ci % 2]
        for r in range(3):
            pltpu.make_async_copy(fidx_hbm.at[pl.ds(0, _BS)],
                                  b.at[pl.ds(r * _BS, _BS)],
                                  semi[ci % 2]).wait()

    fire_idx(0)
    for ci in range(bpt):
        blk = blk0 + ci
        valid = blk < nblk
        if ci + 1 < bpt:
            fire_idx(ci + 1)
        if ci - 2 >= 0:
            @pl.when((blk0 + ci - 2) < nblk)
            def _():
                pltpu.make_async_copy(
                    out_hbm.at[pl.ds(0, _BS * 16)], stgb[ci % 2],
                    semo[ci % 2]).wait()
        wait_idx(ci)
        ib = idxb[ci % 2]
        stg = stgb[ci % 2]

        @plsc.parallel_loop(0, _BS // L, unroll=4)
        def group(g):
            daddr = ((g % 4) * 16 + lane) * 128 + 16 * (g // 4)
            for r in range(3):
                iv = ib[pl.ds(r * _BS + g * L, L)]
                iv4 = iv * 4
                for c in range(3):
                    v = plsc.load_gather(pos_v, [iv4 + c])
                    plsc.store_scatter(stg, [daddr + (3 * r + c)], v)

        @pl.when(valid)
        def _():
            pltpu.async_copy(stg, out_hbm.at[pl.ds(blk * (_BS * 16),
                                                   _BS * 16)], semo[ci % 2])
    for ci in (bpt - 2, bpt - 1):
        @pl.when((blk0 + ci) < nblk)
        def _():
            pltpu.make_async_copy(out_hbm.at[pl.ds(0, _BS * 16)],
                                  stgb[ci % 2], semo[ci % 2]).wait()


def _sc_gather(pos4_flat, fidx, n_nodes, n_faces):
    nblk = n_faces // _BS
    bpt = (nblk + NW - 1) // NW
    k = pl.kernel(
        functools.partial(_gather_body, n_nodes, n_faces, nblk, bpt),
        out_type=jax.ShapeDtypeStruct((n_faces * 16,), jnp.float32),
        mesh=_MESH,
        scratch_types=[
            pltpu.VMEM((4 * n_nodes,), jnp.float32),
            pltpu.VMEM((3 * _BS,), jnp.int32),
            pltpu.VMEM((3 * _BS,), jnp.int32),
            pltpu.VMEM((_BS * 16,), jnp.float32),
            pltpu.VMEM((_BS * 16,), jnp.float32),
            pltpu.SemaphoreType.DMA,
            pltpu.SemaphoreType.DMA,
            pltpu.SemaphoreType.DMA,
            pltpu.SemaphoreType.DMA,
        ],
        compiler_params=_SC_PARAMS,
    )
    return k(pos4_flat, fidx)


def _mlp_body(ft, pp_ref, ff_ref, a_ref, w1f_ref, b1_ref, w2_ref, b2_ref,
              w2t_ref, b2t_ref, nff_ref, *cor_refs):
    ff = ff_ref[...].astype(jnp.bfloat16)
    base = jnp.dot(ff, w1f_ref[...], preferred_element_type=jnp.float32)
    base = base + b1_ref[...]
    pt = jnp.dot(pp_ref[...].astype(jnp.bfloat16), a_ref[...],
                 preferred_element_type=jnp.float32)
    hs = []
    for r in range(3):
        for j in range(ft // _BS):
            for k in range(8):
                hs.append(base[_BS * j + 64 * k:_BS * j + 64 * (k + 1)] +
                          pt[64 * j:64 * (j + 1),
                             384 * k + 128 * r:384 * k + 128 * r + 128])
    h3 = jnp.maximum(jnp.concatenate(hs, axis=0), 0.0).astype(jnp.bfloat16)
    g3 = jnp.dot(h3, w2_ref[...], preferred_element_type=jnp.float32)
    g3 = g3 + b2_ref[...]
    cpt = lax.dot_general(w2t_ref[...], h3, (((1,), (1,)), ((), ())),
                          preferred_element_type=jnp.float32)
    cpt = cpt + b2t_ref[...]
    i4 = pl.program_id(0) % 4
    for r in range(3):
        for c in range(3):
            cor_refs[3 * r + c][pl.ds(i4 * ft, ft)] = \
                cpt[c, r * ft:(r + 1) * ft]
    acc = g3[0:ft, 3:] + g3[ft:2 * ft, 3:] + g3[2 * ft:3 * ft, 3:]
    nff_ref[...] = acc * (1.0 / 3.0)


def _tc_mlp(pp, ff, a_stack, w1f, b1, w2, b2, w2t, b2t, n_faces, out_ch):
    ft = 2560
    assert n_faces % ft == 0
    grid = n_faces // ft
    return pl.pallas_call(
        functools.partial(_mlp_body, ft),
        grid=(grid,),
        in_specs=[
            pl.BlockSpec((ft * 16 // 128, 128), lambda i: (i, 0)),
            pl.BlockSpec((ft, 128), lambda i: (i, 0)),
            pl.BlockSpec((128, 3072), lambda i: (0, 0)),
            pl.BlockSpec((128, 128), lambda i: (0, 0)),
            pl.BlockSpec((1, 128), lambda i: (0, 0)),
            pl.BlockSpec((128, 32), lambda i: (0, 0)),
            pl.BlockSpec((1, 32), lambda i: (0, 0)),
            pl.BlockSpec((3, 128), lambda i: (0, 0)),
            pl.BlockSpec((3, 1), lambda i: (0, 0)),
        ],
        out_specs=[pl.BlockSpec((ft, out_ch), lambda i: (i, 0))] +
                  [pl.BlockSpec((4 * ft,), lambda i: (i // 4,))
                   for _ in range(9)],
        out_shape=[jax.ShapeDtypeStruct((n_faces, out_ch), jnp.float32)] +
                  [jax.ShapeDtypeStruct((4 * ft * ((grid + 3) // 4),),
                                        jnp.float32) for _ in range(9)],
    )(pp, ff, a_stack, w1f, b1, w2, b2, w2t, b2t)


def _scatter_body(npad, n_faces, fpt, ch, idx_hbm, c00, c01, c02, c10, c11,
                  c12, c20, c21, c22, part_hbm,
                  acc_v, idxb0, idxb1, buf0, buf1, tmp0, tmp1, shared_v,
                  sem_b0, sem_b1, sem_r0, sem_r1):
    cid = lax.axis_index("c")
    sid = lax.axis_index("s")
    wid = sid * NC + cid
    fbase0 = wid * fpt
    ones = jnp.ones((L,), jnp.float32)

    nwords = npad * 4

    @plsc.parallel_loop(0, nwords // L, unroll=4)
    def zero(i):
        acc_v[pl.ds(i * L, L)] = jnp.zeros((L,), jnp.float32)

    carr = ((c00, c01, c02), (c10, c11, c12), (c20, c21, c22))
    chunks = [(r, k) for r in range(3) for k in range(fpt // ch)]
    idxbs = (idxb0, idxb1)
    bufs = (buf0, buf1)
    sems = (sem_b0, sem_b1)

    def fire(ci):
        r, k = chunks[ci]
        fb = fbase0 + k * ch
        ib, b, s = idxbs[ci % 2], bufs[ci % 2], sems[ci % 2]
        pltpu.async_copy(idx_hbm.at[pl.ds(r * n_faces + fb, ch)], ib, s)
        for c in range(3):
            pltpu.async_copy(carr[r][c].at[pl.ds(fb, ch)],
                             b.at[pl.ds(c * ch, ch)], s)

    def wait(ci):
        ib, b, s = idxbs[ci % 2], bufs[ci % 2], sems[ci % 2]
        pltpu.make_async_copy(idx_hbm.at[pl.ds(0, ch)], ib, s).wait()
        for c in range(3):
            pltpu.make_async_copy(c00.at[pl.ds(0, ch)],
                                  b.at[pl.ds(c * ch, ch)], s).wait()

    fire(0)
    for ci in range(len(chunks)):
        if ci + 1 < len(chunks):
            fire(ci + 1)
        wait(ci)
        ib, b = idxbs[ci % 2], bufs[ci % 2]

        @plsc.parallel_loop(0, ch // L, unroll=4)
        def group(g):
            iv = ib[pl.ds(g * L, L)]
            iv4 = iv * 4
            plsc.addupdate_scatter(acc_v, [iv4], b[pl.ds(g * L, L)])
            plsc.addupdate_scatter(acc_v, [iv4 + 1],
                                   b[pl.ds(ch + g * L, L)])
            plsc.addupdate_scatter(acc_v, [iv4 + 2],
                                   b[pl.ds(2 * ch + g * L, L)])
            plsc.addupdate_scatter(acc_v, [iv4 + 3], ones)

    pltpu.sync_copy(acc_v, shared_v.at[pl.ds(sid * nwords, nwords)])
    plsc.subcore_barrier()

    wpt = nwords // NS
    off = sid * wpt
    red = acc_v
    tb = (tmp0, tmp1)
    ts = (sem_r0, sem_r1)
    pltpu.sync_copy(shared_v.at[pl.ds(off, wpt)], red.at[pl.ds(0, wpt)])
    pltpu.async_copy(shared_v.at[pl.ds(nwords + off, wpt)], tmp1, sem_r1)
    for t in range(1, NS):
        if t + 1 < NS:
            pltpu.async_copy(shared_v.at[pl.ds((t + 1) * nwords + off, wpt)],
                             tb[(t + 1) % 2], ts[(t + 1) % 2])
        pltpu.make_async_copy(part_hbm.at[pl.ds(0, wpt)], tb[t % 2],
                              ts[t % 2]).wait()
        tv = tb[t % 2]

        @plsc.parallel_loop(0, wpt // L, unroll=4)
        def add(i):
            red[pl.ds(i * L, L)] = red[pl.ds(i * L, L)] + tv[pl.ds(i * L, L)]
    pltpu.sync_copy(red.at[pl.ds(0, wpt)],
                    part_hbm.at[pl.ds(cid * nwords + off, wpt)])


def _sc_scatter(idx_flat, comps, npad, n_faces):
    fpt = n_faces // NW
    ch = 2000
    assert fpt % ch == 0
    nwords = npad * 4
    k = pl.kernel(
        functools.partial(_scatter_body, npad, n_faces, fpt, ch),
        out_type=jax.ShapeDtypeStruct((NC * nwords,), jnp.float32),
        mesh=_MESH,
        scratch_types=[
            pltpu.VMEM((nwords,), jnp.float32),
            pltpu.VMEM((ch,), jnp.int32),
            pltpu.VMEM((ch,), jnp.int32),
            pltpu.VMEM((3 * ch,), jnp.float32),
            pltpu.VMEM((3 * ch,), jnp.float32),
            pltpu.VMEM((nwords // NS,), jnp.float32),
            pltpu.VMEM((nwords // NS,), jnp.float32),
            pltpu.VMEM_SHARED((NS * nwords,), jnp.float32),
            pltpu.SemaphoreType.DMA,
            pltpu.SemaphoreType.DMA,
            pltpu.SemaphoreType.DMA,
            pltpu.SemaphoreType.DMA,
        ],
        compiler_params=_SC_PARAMS,
    )
    return k(idx_flat, *comps)


def _fin_body(n_nodes, part_ref, pos_ref, dpos_ref, npos_ref):
    s = part_ref[0] + part_ref[1]
    sums = s[:n_nodes, 0:3]
    cnt = s[:n_nodes, 3:4]
    delta = sums / jnp.maximum(cnt, 1.0)
    dpos_ref[...] = delta
    npos_ref[...] = pos_ref[...] + delta


def _tc_finalize(part, pos, n_nodes, npad):
    return pl.pallas_call(
        functools.partial(_fin_body, n_nodes),
        in_specs=[
            pl.BlockSpec((2, npad, 4), lambda: (0, 0, 0)),
            pl.BlockSpec((n_nodes, 3), lambda: (0, 0)),
        ],
        out_specs=[
            pl.BlockSpec((n_nodes, 3), lambda: (0, 0)),
            pl.BlockSpec((n_nodes, 3), lambda: (0, 0)),
        ],
        out_shape=[
            jax.ShapeDtypeStruct((n_nodes, 3), jnp.float32),
            jax.ShapeDtypeStruct((n_nodes, 3), jnp.float32),
        ],
    )(part, pos)


def kernel(pos, faces, face_features, W1, b1, W2, b2):
    n_nodes, _ = pos.shape
    n_faces = faces.shape[0]
    out_ch = W2.shape[1] - 3

    fidx = faces.astype(jnp.int32).T.reshape(-1)
    pos4 = jnp.pad(pos, ((0, 0), (0, 1))).reshape(-1)

    p16_flat = _sc_gather(pos4, fidx, n_nodes, n_faces)
    pp = p16_flat.reshape(n_faces * 16 // 128, 128)

    wa, wb = W1[0:3], W1[3:6]
    s = wa + wb
    a0 = jnp.concatenate([-s, wa, wb], axis=0)
    a1 = jnp.concatenate([wb, -s, wa], axis=0)
    a2 = jnp.concatenate([wa, wb, -s], axis=0)
    a16 = jnp.pad(jnp.concatenate([a0, a1, a2], axis=1), ((0, 7), (0, 0)))
    a_stack = jnp.kron(jnp.eye(8, dtype=jnp.float32), a16)

    bf = jnp.bfloat16
    outs = _tc_mlp(pp, face_features, a_stack.astype(bf), W1[6:].astype(bf),
                   b1[None], W2.astype(bf), b2[None],
                   W2[:, 0:3].T.astype(bf), b2[0:3].reshape(3, 1),
                   n_faces, out_ch)
    nff, comps = outs[0], outs[1:]

    npad = NS * ((n_nodes + NS * 8 - 1) // (NS * 8)) * 8
    part = _sc_scatter(fidx, comps, npad, n_faces)

    dpos, npos = _tc_finalize(part.reshape(NC, npad, 4), pos, n_nodes, npad)
    return (dpos, npos, nff)
